# Initial kernel scaffold; baseline (speedup 1.0000x reference)
import jax
import jax.numpy as jnp
from jax.experimental import pallas as pl


def kernel(x, edge_index, W1, b1, W2, b2, W3, b3):
    raise NotImplementedError("write your pallas kernel here")


# speedup vs baseline: 16.1113x; 16.1113x over previous
import functools

import jax
import jax.numpy as jnp
from jax import lax
from jax.experimental import pallas as pl
from jax.experimental.pallas import tpu as pltpu
from jax.experimental.pallas import tpu_sc as plsc

N_NODES = 10000
N_EDGES = 320000
NC, NS = 2, 16
NW = NC * NS
LANES = 16

R_MAIN = 624
R_LAST = N_NODES - R_MAIN * (NS - 1)
EDGES_PER_TILE = N_EDGES // NW
K_CHUNK = 128
N_FULL = EDGES_PER_TILE // K_CHUNK
K_TAIL = EDGES_PER_TILE - N_FULL * K_CHUNK


def _sc_mesh():
    return plsc.VectorSubcoreMesh(
        core_axis_name="c", subcore_axis_name="s",
        num_cores=NC, num_subcores=NS)


def _make_deg_kernel():
    CH = 2000
    NCH = EDGES_PER_TILE // CH

    @functools.partial(
        pl.kernel,
        out_type=jax.ShapeDtypeStruct((NW, N_NODES), jnp.float32),
        mesh=_sc_mesh(),
        compiler_params=pltpu.CompilerParams(needs_layout_passes=False),
        scratch_types=[
            pltpu.VMEM((N_NODES,), jnp.float32),
            pltpu.VMEM((CH,), jnp.int32),
        ],
    )
    def deg_kernel(dst_hbm, degp_hbm, deg_v, idx_v):
        cid = lax.axis_index("c")
        sid = lax.axis_index("s")
        wid = cid * NS + sid

        def zero_body(j, carry):
            deg_v[pl.ds(j * LANES, LANES)] = jnp.zeros((LANES,), jnp.float32)
            return carry
        lax.fori_loop(0, N_NODES // LANES, zero_body, 0)

        base0 = wid * EDGES_PER_TILE
        ones = jnp.ones((LANES,), jnp.float32)

        def ch_body(c, carry):
            pltpu.sync_copy(dst_hbm.at[pl.ds(base0 + c * CH, CH)], idx_v)

            def v_body(j, inner):
                v = idx_v[pl.ds(j * LANES, LANES)]
                plsc.addupdate_scatter(deg_v, [v], ones)
                return inner
            lax.fori_loop(0, CH // LANES, v_body, 0)
            return carry
        lax.fori_loop(0, NCH, ch_body, 0)

        pltpu.sync_copy(deg_v, degp_hbm.at[wid])

    return deg_kernel


def _make_scatter_kernel(F):
    @functools.partial(
        pl.kernel,
        out_type=jax.ShapeDtypeStruct((NC, N_NODES, F), jnp.float32),
        mesh=_sc_mesh(),
        compiler_params=pltpu.CompilerParams(
            needs_layout_passes=False,
            use_tc_tiling_on_sc=(F % 128 == 0)),
        scratch_types=[
            pltpu.VMEM_SHARED((N_NODES, F), jnp.float32),
            pltpu.VMEM((K_CHUNK,), jnp.int32),
            pltpu.VMEM((K_CHUNK,), jnp.int32),
            pltpu.VMEM((K_CHUNK, F), jnp.float32),
            pltpu.VMEM((K_TAIL,), jnp.int32),
            pltpu.VMEM((K_TAIL,), jnp.int32),
            pltpu.VMEM((K_TAIL, F), jnp.float32),
            pltpu.SemaphoreType.DMA,
        ],
    )
    def scatter_kernel(g_hbm, z_hbm, src_hbm, dst_hbm, out_hbm,
                       acc, sidx, didx, rows, sidx_t, didx_t, rows_t, sem):
        cid = lax.axis_index("c")
        sid = lax.axis_index("s")
        wid = cid * NS + sid

        def striped(fn):
            @pl.when(sid < NS - 1)
            def _():
                fn(sid * R_MAIN, R_MAIN)

            @pl.when(sid == NS - 1)
            def _():
                fn((NS - 1) * R_MAIN, R_LAST)

        def init_stripe(off, size):
            @pl.when(cid == 0)
            def _():
                pltpu.sync_copy(g_hbm.at[pl.ds(off, size)],
                                acc.at[pl.ds(off, size)])

            @pl.when(cid != 0)
            def _():
                pltpu.sync_copy(z_hbm.at[pl.ds(off, size)],
                                acc.at[pl.ds(off, size)])

        striped(init_stripe)
        plsc.subcore_barrier()

        base0 = wid * EDGES_PER_TILE

        def ch_body(c, carry):
            b = base0 + c * K_CHUNK
            pltpu.sync_copy(src_hbm.at[pl.ds(b, K_CHUNK)], sidx)
            pltpu.sync_copy(dst_hbm.at[pl.ds(b, K_CHUNK)], didx)
            pltpu.async_copy(g_hbm.at[sidx], rows, sem).wait()
            pltpu.sync_copy(rows, acc.at[didx], add=True)
            return carry
        lax.fori_loop(0, N_FULL, ch_body, 0)

        b = base0 + N_FULL * K_CHUNK
        pltpu.sync_copy(src_hbm.at[pl.ds(b, K_TAIL)], sidx_t)
        pltpu.sync_copy(dst_hbm.at[pl.ds(b, K_TAIL)], didx_t)
        pltpu.async_copy(g_hbm.at[sidx_t], rows_t, sem).wait()
        pltpu.sync_copy(rows_t, acc.at[didx_t], add=True)

        plsc.subcore_barrier()

        def write_stripe(off, size):
            pltpu.sync_copy(acc.at[pl.ds(off, size)],
                            out_hbm.at[cid, pl.ds(off, size)])

        striped(write_stripe)

    return scatter_kernel


def _dinv_body(degp_ref, dinv_ref):
    deg = jnp.sum(degp_ref[...], axis=0) + 1.0
    dinv_ref[...] = jnp.broadcast_to(lax.rsqrt(deg)[:, None], dinv_ref.shape)


def _dinv_kernel(degp):
    return pl.pallas_call(
        _dinv_body,
        out_shape=jax.ShapeDtypeStruct((N_NODES, 128), jnp.float32),
        grid=(1,),
        in_specs=[pl.BlockSpec((NW, N_NODES), lambda i: (0, 0))],
        out_specs=pl.BlockSpec((N_NODES, 128), lambda i: (0, 0)),
    )(degp)


_RB = 1000
_NRB = N_NODES // _RB


def _mm_first_body(x_ref, w_ref, dinv_ref, o_ref):
    h = jnp.dot(x_ref[...], w_ref[...], preferred_element_type=jnp.float32)
    o_ref[...] = dinv_ref[...] * h


def _mm_first(x, W, dinvb):
    return pl.pallas_call(
        _mm_first_body,
        out_shape=jax.ShapeDtypeStruct((N_NODES, W.shape[1]), jnp.float32),
        grid=(_NRB,),
        in_specs=[
            pl.BlockSpec((_RB, 128), lambda i: (i, 0)),
            pl.BlockSpec(W.shape, lambda i: (0, 0)),
            pl.BlockSpec((_RB, 128), lambda i: (i, 0)),
        ],
        out_specs=pl.BlockSpec((_RB, W.shape[1]), lambda i: (i, 0)),
    )(x, W, dinvb)


def _mm_mid_body(p_ref, dinv_ref, b_ref, w_ref, o_ref):
    s = p_ref[0] + p_ref[1]
    xin = jnp.maximum(dinv_ref[...] * s + b_ref[...], 0.0)
    h = jnp.dot(xin, w_ref[...], preferred_element_type=jnp.float32)
    o_ref[...] = dinv_ref[:, : o_ref.shape[1]] * h


def _mm_mid(p, dinvb, b, W):
    Fo = W.shape[1]
    return pl.pallas_call(
        _mm_mid_body,
        out_shape=jax.ShapeDtypeStruct((N_NODES, Fo), jnp.float32),
        grid=(_NRB,),
        in_specs=[
            pl.BlockSpec((NC, _RB, 128), lambda i: (0, i, 0)),
            pl.BlockSpec((_RB, 128), lambda i: (i, 0)),
            pl.BlockSpec((1, 128), lambda i: (0, 0)),
            pl.BlockSpec(W.shape, lambda i: (0, 0)),
        ],
        out_specs=pl.BlockSpec((_RB, Fo), lambda i: (i, 0)),
    )(p, dinvb, b, W)


def _final_body(p_ref, dinv_ref, b_ref, o_ref):
    s = p_ref[0] + p_ref[1]
    o_ref[...] = jnp.maximum(dinv_ref[:, : o_ref.shape[1]] * s + b_ref[...], 0.0)


def _final(p, dinvb, b3):
    Fo = p.shape[2]
    return pl.pallas_call(
        _final_body,
        out_shape=jax.ShapeDtypeStruct((N_NODES, Fo), jnp.float32),
        grid=(_NRB,),
        in_specs=[
            pl.BlockSpec((NC, _RB, Fo), lambda i: (0, i, 0)),
            pl.BlockSpec((_RB, 128), lambda i: (i, 0)),
            pl.BlockSpec((1, Fo), lambda i: (0, 0)),
        ],
        out_specs=pl.BlockSpec((_RB, Fo), lambda i: (i, 0)),
    )(p, dinvb, b3)


def kernel(x, edge_index, W1, b1, W2, b2, W3, b3):
    ei = edge_index.astype(jnp.int32)
    src = ei[0]
    dst = ei[1]
    z128 = jnp.zeros((N_NODES, 128), jnp.float32)
    z16 = jnp.zeros((N_NODES, 16), jnp.float32)
    b1r = b1.reshape(1, 128)
    b2r = b2.reshape(1, 128)
    b3r = b3.reshape(1, 16)

    deg_k = _make_deg_kernel()
    scat128 = _make_scatter_kernel(128)
    scat16 = _make_scatter_kernel(16)

    degp = deg_k(dst)
    dinvb = _dinv_kernel(degp)

    g1 = _mm_first(x, W1, dinvb)
    p1 = scat128(g1, z128, src, dst)
    g2 = _mm_mid(p1, dinvb, b1r, W2)
    p2 = scat128(g2, z128, src, dst)
    g3 = _mm_mid(p2, dinvb, b2r, W3)
    p3 = scat16(g3, z16, src, dst)
    return _final(p3, dinvb, b3r)


# speedup vs baseline: 31.3751x; 1.9474x over previous
import functools

import jax
import jax.numpy as jnp
from jax import lax
from jax.experimental import pallas as pl
from jax.experimental.pallas import tpu as pltpu
from jax.experimental.pallas import tpu_sc as plsc

N_NODES = 10000
N_EDGES = 320000
NC, NS = 2, 16
NW = NC * NS
LANES = 16

R_MAIN = 624
R_LAST = N_NODES - R_MAIN * (NS - 1)
EDGES_PER_TILE = N_EDGES // NW
K_CHUNK = 128
N_FULL = EDGES_PER_TILE // K_CHUNK
K_TAIL = EDGES_PER_TILE - N_FULL * K_CHUNK


def _sc_mesh():
    return plsc.VectorSubcoreMesh(
        core_axis_name="c", subcore_axis_name="s",
        num_cores=NC, num_subcores=NS)


def _make_deg_kernel():
    CH = 2000
    NCH = EDGES_PER_TILE // CH

    @functools.partial(
        pl.kernel,
        out_type=jax.ShapeDtypeStruct((NW, N_NODES), jnp.float32),
        mesh=_sc_mesh(),
        compiler_params=pltpu.CompilerParams(needs_layout_passes=False),
        scratch_types=[
            pltpu.VMEM((N_NODES,), jnp.float32),
            pltpu.VMEM((CH,), jnp.int32),
        ],
    )
    def deg_kernel(dst_hbm, degp_hbm, deg_v, idx_v):
        cid = lax.axis_index("c")
        sid = lax.axis_index("s")
        wid = cid * NS + sid

        def zero_body(j, carry):
            deg_v[pl.ds(j * LANES, LANES)] = jnp.zeros((LANES,), jnp.float32)
            return carry
        lax.fori_loop(0, N_NODES // LANES, zero_body, 0)

        base0 = wid * EDGES_PER_TILE
        ones = jnp.ones((LANES,), jnp.float32)

        def ch_body(c, carry):
            pltpu.sync_copy(dst_hbm.at[pl.ds(base0 + c * CH, CH)], idx_v)

            def v_body(j, inner):
                v = idx_v[pl.ds(j * LANES, LANES)]
                plsc.addupdate_scatter(deg_v, [v], ones)
                return inner
            lax.fori_loop(0, CH // LANES, v_body, 0)
            return carry
        lax.fori_loop(0, NCH, ch_body, 0)

        pltpu.sync_copy(deg_v, degp_hbm.at[wid])

    return deg_kernel


def _make_scatter_kernel(F):
    K = K_CHUNK
    CPT = 78
    E_MAIN = NW * CPT * K
    XCH = (N_EDGES - E_MAIN) // K
    NBUF = 2
    NGRP = CPT // NBUF

    @functools.partial(
        pl.kernel,
        out_type=jax.ShapeDtypeStruct((NC, N_NODES, F), jnp.float32),
        mesh=_sc_mesh(),
        compiler_params=pltpu.CompilerParams(
            needs_layout_passes=False,
            use_tc_tiling_on_sc=(F % 128 == 0)),
        scratch_types=[
            pltpu.VMEM_SHARED((N_NODES, F), jnp.float32),
            pltpu.VMEM((NBUF, K), jnp.int32),
            pltpu.VMEM((NBUF, K), jnp.int32),
            pltpu.VMEM((NBUF, K, F), jnp.float32),
            pltpu.SemaphoreType.DMA,
            pltpu.SemaphoreType.DMA,
            pltpu.SemaphoreType.DMA,
            pltpu.SemaphoreType.DMA,
            pltpu.SemaphoreType.DMA,
            pltpu.SemaphoreType.DMA,
            pltpu.SemaphoreType.DMA,
            pltpu.SemaphoreType.DMA,
        ],
    )
    def scatter_kernel(g_hbm, z_hbm, src_hbm, dst_hbm, out_hbm,
                       acc, sidx, didx, rows,
                       sg0, sg1, ss0, ss1, si0, si1, sd0, sd1):
        semg = (sg0, sg1)
        sems = (ss0, ss1)
        semis = (si0, si1)
        semid = (sd0, sd1)
        cid = lax.axis_index("c")
        sid = lax.axis_index("s")
        wid = cid * NS + sid
        base0 = wid * (CPT * K)

        def striped(fn):
            @pl.when(sid < NS - 1)
            def _():
                fn(sid * R_MAIN, R_MAIN)

            @pl.when(sid == NS - 1)
            def _():
                fn((NS - 1) * R_MAIN, R_LAST)

        def init_stripe(off, size):
            @pl.when(cid == 0)
            def _():
                pltpu.sync_copy(g_hbm.at[pl.ds(off, size)],
                                acc.at[pl.ds(off, size)])

            @pl.when(cid != 0)
            def _():
                pltpu.sync_copy(z_hbm.at[pl.ds(off, size)],
                                acc.at[pl.ds(off, size)])

        striped(init_stripe)

        def istart_s(j, b):
            pltpu.async_copy(src_hbm.at[pl.ds(base0 + j * K, K)],
                             sidx.at[b], semis[b])

        def iwait_s(j, b):
            pltpu.make_async_copy(src_hbm.at[pl.ds(base0 + j * K, K)],
                                  sidx.at[b], semis[b]).wait()

        def istart_d(j, b):
            pltpu.async_copy(dst_hbm.at[pl.ds(base0 + j * K, K)],
                             didx.at[b], semid[b])

        def iwait_d(j, b):
            pltpu.make_async_copy(dst_hbm.at[pl.ds(base0 + j * K, K)],
                                  didx.at[b], semid[b]).wait()

        def gstart(j, b):
            pltpu.async_copy(g_hbm.at[sidx.at[b]], rows.at[b], semg[b])

        def gwait(j, b):
            pltpu.make_async_copy(g_hbm.at[sidx.at[b]],
                                  rows.at[b], semg[b]).wait()

        def sstart(j, b):
            pltpu.async_copy(rows.at[b], acc.at[didx.at[b]],
                             sems[b], add=True)

        def swait(j, b):
            pltpu.make_async_copy(rows.at[b], acc.at[didx.at[b]],
                                  sems[b]).wait()

        plsc.subcore_barrier()

        for b in range(NBUF):
            istart_s(b, b)
            istart_d(b, b)
        for b in range(NBUF):
            iwait_s(b, b)
            gstart(b, b)

        def group(gi, carry):
            j0 = gi * NBUF
            for b in range(NBUF):
                j = j0 + b
                gwait(j, b)

                @pl.when(gi < NGRP - 1)
                def _():
                    istart_s(j + NBUF, b)

                iwait_d(j, b)
                sstart(j, b)
                swait(j, b)

                @pl.when(gi < NGRP - 1)
                def _():
                    istart_d(j + NBUF, b)
                    iwait_s(j + NBUF, b)
                    gstart(j + NBUF, b)
            return carry
        lax.fori_loop(0, NGRP, group, 0)

        @pl.when(wid < XCH)
        def _():
            bx = E_MAIN + wid * K - base0
            istart_s(bx // K, 0)
            istart_d(bx // K, 0)
            iwait_s(bx // K, 0)
            iwait_d(bx // K, 0)
            pltpu.async_copy(g_hbm.at[sidx.at[0]], rows.at[0], semg[0]).wait()
            pltpu.sync_copy(rows.at[0], acc.at[didx.at[0]], add=True)

        plsc.subcore_barrier()

        def write_stripe(off, size):
            pltpu.sync_copy(acc.at[pl.ds(off, size)],
                            out_hbm.at[cid, pl.ds(off, size)])

        striped(write_stripe)

    return scatter_kernel


def _dinv_body(degp_ref, dinv_ref):
    deg = jnp.sum(degp_ref[...], axis=0) + 1.0
    dinv_ref[...] = jnp.broadcast_to(lax.rsqrt(deg)[:, None], dinv_ref.shape)


def _dinv_kernel(degp):
    return pl.pallas_call(
        _dinv_body,
        out_shape=jax.ShapeDtypeStruct((N_NODES, 128), jnp.float32),
        grid=(1,),
        in_specs=[pl.BlockSpec((NW, N_NODES), lambda i: (0, 0))],
        out_specs=pl.BlockSpec((N_NODES, 128), lambda i: (0, 0)),
    )(degp)


_RB = 1000
_NRB = N_NODES // _RB


def _mm_first_body(x_ref, w_ref, dinv_ref, o_ref):
    h = jnp.dot(x_ref[...], w_ref[...], preferred_element_type=jnp.float32)
    o_ref[...] = dinv_ref[...] * h


def _mm_first(x, W, dinvb):
    return pl.pallas_call(
        _mm_first_body,
        out_shape=jax.ShapeDtypeStruct((N_NODES, W.shape[1]), jnp.float32),
        grid=(_NRB,),
        in_specs=[
            pl.BlockSpec((_RB, 128), lambda i: (i, 0)),
            pl.BlockSpec(W.shape, lambda i: (0, 0)),
            pl.BlockSpec((_RB, 128), lambda i: (i, 0)),
        ],
        out_specs=pl.BlockSpec((_RB, W.shape[1]), lambda i: (i, 0)),
    )(x, W, dinvb)


def _mm_mid_body(p_ref, dinv_ref, b_ref, w_ref, o_ref):
    s = p_ref[0] + p_ref[1]
    xin = jnp.maximum(dinv_ref[...] * s + b_ref[...], 0.0)
    h = jnp.dot(xin, w_ref[...], preferred_element_type=jnp.float32)
    o_ref[...] = dinv_ref[:, : o_ref.shape[1]] * h


def _mm_mid(p, dinvb, b, W):
    Fo = W.shape[1]
    return pl.pallas_call(
        _mm_mid_body,
        out_shape=jax.ShapeDtypeStruct((N_NODES, Fo), jnp.float32),
        grid=(_NRB,),
        in_specs=[
            pl.BlockSpec((NC, _RB, 128), lambda i: (0, i, 0)),
            pl.BlockSpec((_RB, 128), lambda i: (i, 0)),
            pl.BlockSpec((1, 128), lambda i: (0, 0)),
            pl.BlockSpec(W.shape, lambda i: (0, 0)),
        ],
        out_specs=pl.BlockSpec((_RB, Fo), lambda i: (i, 0)),
    )(p, dinvb, b, W)


def _final_body(p_ref, dinv_ref, b_ref, o_ref):
    s = p_ref[0] + p_ref[1]
    o_ref[...] = jnp.maximum(dinv_ref[:, : o_ref.shape[1]] * s + b_ref[...], 0.0)


def _final(p, dinvb, b3):
    Fo = p.shape[2]
    return pl.pallas_call(
        _final_body,
        out_shape=jax.ShapeDtypeStruct((N_NODES, Fo), jnp.float32),
        grid=(_NRB,),
        in_specs=[
            pl.BlockSpec((NC, _RB, Fo), lambda i: (0, i, 0)),
            pl.BlockSpec((_RB, 128), lambda i: (i, 0)),
            pl.BlockSpec((1, Fo), lambda i: (0, 0)),
        ],
        out_specs=pl.BlockSpec((_RB, Fo), lambda i: (i, 0)),
    )(p, dinvb, b3)


def kernel(x, edge_index, W1, b1, W2, b2, W3, b3):
    ei = edge_index.astype(jnp.int32)
    src = ei[0]
    dst = ei[1]
    z128 = jnp.zeros((N_NODES, 128), jnp.float32)
    z16 = jnp.zeros((N_NODES, 16), jnp.float32)
    b1r = b1.reshape(1, 128)
    b2r = b2.reshape(1, 128)
    b3r = b3.reshape(1, 16)

    deg_k = _make_deg_kernel()
    scat128 = _make_scatter_kernel(128)
    scat16 = _make_scatter_kernel(16)

    degp = deg_k(dst)
    dinvb = _dinv_kernel(degp)

    g1 = _mm_first(x, W1, dinvb)
    p1 = scat128(g1, z128, src, dst)
    g2 = _mm_mid(p1, dinvb, b1r, W2)
    p2 = scat128(g2, z128, src, dst)
    g3 = _mm_mid(p2, dinvb, b2r, W3)
    p3 = scat16(g3, z16, src, dst)
    return _final(p3, dinvb, b3r)
